# Initial kernel scaffold; baseline (speedup 1.0000x reference)
import jax
import jax.numpy as jnp
from jax.experimental import pallas as pl


def kernel(labels, embeddings):
    raise NotImplementedError("write your pallas kernel here")


# speedup vs baseline: 9.7871x; 9.7871x over previous
import functools

import jax
import jax.numpy as jnp
from jax import lax
from jax.experimental import pallas as pl
from jax.experimental.pallas import tpu as pltpu
from jax.experimental.pallas import tpu_sc as plsc

D = 64
CH = 128
GC = 4
GR = CH * GC

_info = plsc.get_sparse_core_info()
NW = _info.num_cores * _info.num_subcores


def _sc_gather(table_hbm, idx_hbm, out_hbm, idx_a, idx_b, buf_a, buf_b,
               gsem_a, gsem_b, ssem_a, ssem_b, *, n_groups):
    wid = lax.axis_index("s") * _info.num_cores + lax.axis_index("c")
    wbase = wid * (n_groups * GR)

    def fire_group(g, idx_v, buf, gsem):
        pltpu.sync_copy(idx_hbm.at[wid, g], idx_v)
        for b in range(GC):
            pltpu.async_copy(
                table_hbm.at[idx_v.at[b]],
                buf.at[pl.ds(b * CH, CH)],
                gsem,
            )

    def wait_gathers(buf, gsem):
        pltpu.make_async_copy(table_hbm.at[pl.ds(0, GR)], buf, gsem).wait()

    def fire_store(g, buf, ssem):
        pltpu.async_copy(buf, out_hbm.at[pl.ds(wbase + g * GR, GR)], ssem)

    def wait_store(g, buf, ssem):
        pltpu.make_async_copy(
            buf, out_hbm.at[pl.ds(wbase + g * GR, GR)], ssem).wait()

    fire_group(0, idx_a, buf_a, gsem_a)

    def body(i, _):
        g0 = 2 * i
        g1 = g0 + 1

        @pl.when(i > 0)
        def _():
            wait_store(g0 - 1, buf_b, ssem_b)

        fire_group(g1, idx_b, buf_b, gsem_b)
        wait_gathers(buf_a, gsem_a)
        fire_store(g0, buf_a, ssem_a)

        @pl.when(i < (n_groups // 2 - 1))
        def _():
            wait_store(g0, buf_a, ssem_a)
            fire_group(g0 + 2, idx_a, buf_a, gsem_a)

        wait_gathers(buf_b, gsem_b)
        fire_store(g1, buf_b, ssem_b)
        return 0

    lax.fori_loop(0, n_groups // 2, body, 0)
    wait_store(n_groups - 2, buf_a, ssem_a)
    wait_store(n_groups - 1, buf_b, ssem_b)


def kernel(labels, embeddings):
    n, s = labels.shape
    total = n * s
    assert total % (NW * GR) == 0
    n_groups = total // (NW * GR)

    idx = labels.astype(jnp.int32).reshape(NW, n_groups, GC, CH)

    mesh = plsc.VectorSubcoreMesh(core_axis_name="c", subcore_axis_name="s")
    k = functools.partial(
        pl.kernel,
        mesh=mesh,
        out_type=jax.ShapeDtypeStruct((total, D), jnp.float32),
        scratch_types=[
            pltpu.VMEM((GC, CH), jnp.int32),
            pltpu.VMEM((GC, CH), jnp.int32),
            pltpu.VMEM((GR, D), jnp.float32),
            pltpu.VMEM((GR, D), jnp.float32),
            pltpu.SemaphoreType.DMA,
            pltpu.SemaphoreType.DMA,
            pltpu.SemaphoreType.DMA,
            pltpu.SemaphoreType.DMA,
        ],
        compiler_params=pltpu.CompilerParams(use_tc_tiling_on_sc=False),
    )(functools.partial(_sc_gather, n_groups=n_groups))
    out = k(embeddings, idx)
    return out.reshape(n, s, D)


# speedup vs baseline: 13.1578x; 1.3444x over previous
import functools

import jax
import jax.numpy as jnp
from jax import lax
from jax.experimental import pallas as pl
from jax.experimental.pallas import tpu as pltpu
from jax.experimental.pallas import tpu_sc as plsc

D = 64
CH = 128
GC = 4
GR = CH * GC

_info = plsc.get_sparse_core_info()
NW = _info.num_cores * _info.num_subcores


def _sc_gather(table_hbm, idx_hbm, out_hbm, table_v, idx_a, idx_b, buf_a,
               buf_b, gsem_a, gsem_b, ssem_a, ssem_b, *, n_groups):
    wid = lax.axis_index("s") * _info.num_cores + lax.axis_index("c")
    wbase = wid * (n_groups * GR)

    @pl.when(lax.axis_index("s") == 0)
    def _():
        pltpu.sync_copy(table_hbm, table_v)

    plsc.subcore_barrier()

    def fire_group(g, idx_v, buf, gsem):
        pltpu.sync_copy(idx_hbm.at[wid, g], idx_v)
        for b in range(GC):
            pltpu.async_copy(
                table_v.at[idx_v.at[b]],
                buf.at[pl.ds(b * CH, CH)],
                gsem,
            )

    def wait_gathers(buf, gsem):
        pltpu.make_async_copy(table_hbm.at[pl.ds(0, GR)], buf, gsem).wait()

    def fire_store(g, buf, ssem):
        pltpu.async_copy(buf, out_hbm.at[pl.ds(wbase + g * GR, GR)], ssem)

    def wait_store(g, buf, ssem):
        pltpu.make_async_copy(
            buf, out_hbm.at[pl.ds(wbase + g * GR, GR)], ssem).wait()

    fire_group(0, idx_a, buf_a, gsem_a)

    def body(i, _):
        g0 = 2 * i
        g1 = g0 + 1

        @pl.when(i > 0)
        def _():
            wait_store(g0 - 1, buf_b, ssem_b)

        fire_group(g1, idx_b, buf_b, gsem_b)
        wait_gathers(buf_a, gsem_a)
        fire_store(g0, buf_a, ssem_a)

        @pl.when(i < (n_groups // 2 - 1))
        def _():
            wait_store(g0, buf_a, ssem_a)
            fire_group(g0 + 2, idx_a, buf_a, gsem_a)

        wait_gathers(buf_b, gsem_b)
        fire_store(g1, buf_b, ssem_b)
        return 0

    lax.fori_loop(0, n_groups // 2, body, 0)
    wait_store(n_groups - 2, buf_a, ssem_a)
    wait_store(n_groups - 1, buf_b, ssem_b)


def kernel(labels, embeddings):
    n, s = labels.shape
    total = n * s
    assert total % (NW * GR) == 0
    n_groups = total // (NW * GR)

    idx = labels.astype(jnp.int32).reshape(NW, n_groups, GC, CH)

    mesh = plsc.VectorSubcoreMesh(core_axis_name="c", subcore_axis_name="s")
    k = functools.partial(
        pl.kernel,
        mesh=mesh,
        out_type=jax.ShapeDtypeStruct((total, D), jnp.float32),
        scratch_types=[
            pltpu.VMEM_SHARED((1000, D), jnp.float32),
            pltpu.VMEM((GC, CH), jnp.int32),
            pltpu.VMEM((GC, CH), jnp.int32),
            pltpu.VMEM((GR, D), jnp.float32),
            pltpu.VMEM((GR, D), jnp.float32),
            pltpu.SemaphoreType.DMA,
            pltpu.SemaphoreType.DMA,
            pltpu.SemaphoreType.DMA,
            pltpu.SemaphoreType.DMA,
        ],
        compiler_params=pltpu.CompilerParams(use_tc_tiling_on_sc=False),
    )(functools.partial(_sc_gather, n_groups=n_groups))
    out = k(embeddings, idx)
    return out.reshape(n, s, D)


# speedup vs baseline: 13.2649x; 1.0081x over previous
import functools

import jax
import jax.numpy as jnp
from jax import lax
from jax.experimental import pallas as pl
from jax.experimental.pallas import tpu as pltpu
from jax.experimental.pallas import tpu_sc as plsc

D = 64
S = 50
LR = 16

_info = plsc.get_sparse_core_info()
NW = _info.num_cores * _info.num_subcores


def _sc_gather(table_hbm, labels_hbm, out_hbm, table_v, idx_a, idx_b, buf_a,
               buf_b, gsem_a, gsem_b, ssem_a, ssem_b, *, n_groups):
    wid = lax.axis_index("s") * _info.num_cores + lax.axis_index("c")
    wrow = wid * (n_groups * LR)

    @pl.when(lax.axis_index("s") == 0)
    def _():
        pltpu.sync_copy(table_hbm, table_v)

    plsc.subcore_barrier()

    def fire_group(g, idx_v, buf, gsem):
        pltpu.sync_copy(labels_hbm.at[pl.ds(wrow + g * LR, LR)], idx_v)
        for r in range(LR):
            pltpu.async_copy(table_v.at[idx_v.at[r]], buf.at[r], gsem)

    def wait_gathers(buf, gsem):
        pltpu.make_async_copy(out_hbm.at[pl.ds(0, LR)], buf, gsem).wait()

    def fire_store(g, buf, ssem):
        pltpu.async_copy(buf, out_hbm.at[pl.ds(wrow + g * LR, LR)], ssem)

    def wait_store(g, buf, ssem):
        pltpu.make_async_copy(
            buf, out_hbm.at[pl.ds(wrow + g * LR, LR)], ssem).wait()

    fire_group(0, idx_a, buf_a, gsem_a)

    def body(i, _):
        g0 = 2 * i
        g1 = g0 + 1

        @pl.when(i > 0)
        def _():
            wait_store(g0 - 1, buf_b, ssem_b)

        fire_group(g1, idx_b, buf_b, gsem_b)
        wait_gathers(buf_a, gsem_a)
        fire_store(g0, buf_a, ssem_a)

        @pl.when(i < (n_groups // 2 - 1))
        def _():
            wait_store(g0, buf_a, ssem_a)
            fire_group(g0 + 2, idx_a, buf_a, gsem_a)

        wait_gathers(buf_b, gsem_b)
        fire_store(g1, buf_b, ssem_b)
        return 0

    lax.fori_loop(0, n_groups // 2, body, 0)
    wait_store(n_groups - 2, buf_a, ssem_a)
    wait_store(n_groups - 1, buf_b, ssem_b)


def kernel(labels, embeddings):
    n, s = labels.shape
    assert s == S and n % (NW * LR) == 0
    n_groups = n // (NW * LR)

    idx = labels.astype(jnp.int32)

    mesh = plsc.VectorSubcoreMesh(core_axis_name="c", subcore_axis_name="s")
    k = functools.partial(
        pl.kernel,
        mesh=mesh,
        out_type=jax.ShapeDtypeStruct((n, S, D), jnp.float32),
        scratch_types=[
            pltpu.VMEM_SHARED((1000, D), jnp.float32),
            pltpu.VMEM((LR, S), jnp.int32),
            pltpu.VMEM((LR, S), jnp.int32),
            pltpu.VMEM((LR, S, D), jnp.float32),
            pltpu.VMEM((LR, S, D), jnp.float32),
            pltpu.SemaphoreType.DMA,
            pltpu.SemaphoreType.DMA,
            pltpu.SemaphoreType.DMA,
            pltpu.SemaphoreType.DMA,
        ],
        compiler_params=pltpu.CompilerParams(use_tc_tiling_on_sc=False),
    )(functools.partial(_sc_gather, n_groups=n_groups))
    return k(embeddings, idx)
